# baseline (speedup 1.0000x reference)
import functools

import jax
import jax.numpy as jnp
from jax.experimental import pallas as pl
from jax.experimental.pallas import tpu as pltpu


def _viterbi_kernel(obs_ref, tt_ref, out_ref, bp_ref, ttb_ref, *, N, C, L):
    for p in range(C):
        ttb_ref[p] = jnp.broadcast_to(tt_ref[p][None, :], (N, C))

    def fwd_body(t, fv):

        def sp(p):
            return fv[:, p : p + 1] + ttb_ref[p]

        gvals, gidxs = [], []
        for g in range(C // 16):
            vals, idxs = [], []
            for i in range(8):
                p0 = 16 * g + 2 * i
                a, b = sp(p0), sp(p0 + 1)
                pred = b > a
                vals.append(jnp.maximum(a, b))
                idxs.append(jnp.where(pred, p0 + 1, p0))
            while len(vals) > 1:
                nv, ni = [], []
                for i in range(len(vals) // 2):
                    a, b = vals[2 * i], vals[2 * i + 1]
                    pred = b > a
                    nv.append(jnp.maximum(a, b))
                    ni.append(jnp.where(pred, idxs[2 * i + 1], idxs[2 * i]))
                vals, idxs = nv, ni
            gvals.append(vals[0])
            gidxs.append(idxs[0])
        while len(gvals) > 1:
            nv, ni = [], []
            for i in range(len(gvals) // 2):
                a, b = gvals[2 * i], gvals[2 * i + 1]
                pred = b > a
                nv.append(jnp.maximum(a, b))
                ni.append(jnp.where(pred, gidxs[2 * i + 1], gidxs[2 * i]))
            gvals, gidxs = nv, ni
        bp_ref[t] = gidxs[0]
        return gvals[0] + obs_ref[t]

    fv = jax.lax.fori_loop(0, L, fwd_body, jnp.zeros((N, C), jnp.float32))

    lane = jax.lax.broadcasted_iota(jnp.int32, (N, C), 1)
    m = jnp.max(fv, axis=1, keepdims=True)
    end = jnp.min(jnp.where(fv == m, lane, C), axis=1, keepdims=True)

    def back_body(j, bt):
        for k in range(8):
            t = L - 1 - (8 * j + k)
            bp_t = bp_ref[t]
            sel = jnp.where(lane == bt, bp_t, 0)
            bt = jnp.max(sel, axis=1, keepdims=True)
            out_ref[t] = bt[:, 0]
        return bt

    jax.lax.fori_loop(0, L // 8, back_body, end)


@jax.jit
def kernel(observes, transitions):
    N, C, L = observes.shape
    obs_t = jnp.transpose(observes, (2, 0, 1))
    tt = transitions.T
    path_t = pl.pallas_call(
        functools.partial(_viterbi_kernel, N=N, C=C, L=L),
        out_shape=jax.ShapeDtypeStruct((L, N), jnp.int32),
        in_specs=[
            pl.BlockSpec(memory_space=pltpu.VMEM),
            pl.BlockSpec(memory_space=pltpu.VMEM),
        ],
        out_specs=pl.BlockSpec(memory_space=pltpu.VMEM),
        scratch_shapes=[
            pltpu.VMEM((L, N, C), jnp.int32),
            pltpu.VMEM((C, N, C), jnp.float32),
        ],
    )(obs_t, tt)
    return path_t.T
